# speedup vs baseline: 3.0042x; 3.0042x over previous
import jax
import jax.numpy as jnp
from jax.experimental import pallas as pl

_BLOCK_ROWS = 1024


def _copy_kernel(w_ref, o_ref):
    o_ref[...] = w_ref[...]


def kernel(x, W):
    seq_len = x.shape[1]
    rows, dim = W.shape
    grid = (seq_len // _BLOCK_ROWS,)
    out = pl.pallas_call(
        _copy_kernel,
        grid=grid,
        in_specs=[pl.BlockSpec((_BLOCK_ROWS, dim), lambda i: (i, 0))],
        out_specs=pl.BlockSpec((_BLOCK_ROWS, dim), lambda i: (i, 0)),
        out_shape=jax.ShapeDtypeStruct((seq_len, dim), W.dtype),
    )(W)
    return out[None, :, :]
